# baseline (speedup 1.0000x reference)
import jax
import jax.numpy as jnp
from jax import lax
from jax.experimental import pallas as pl
from jax.experimental.pallas import tpu as pltpu
from jax.experimental.pallas import tpu_sc as plsc

_INV_T = 2.0
_K = 64
_NC, _NS, _L = 2, 16, 16
_NW = _NC * _NS
_C = 384
_CHI = _C // 128
_PIX_W = _K * _CHI * 128
_GRP_W = 8 * _CHI * 128


def _sq_body(x_hbm, tab_hbm, assign_hbm, qst_hbm,
             tab_v, pp_v, qq_v, rr_v, ss_v,
             xg0_v, xg1_v, qg0_v, qg1_v, out0_v, out1_v,
             sem0, sem1, semx0, semx1, semq0, semq1):
    npix = qst_hbm.shape[0] // _C
    grp_per_w = npix // (8 * _NW)
    wid = lax.axis_index("s") * _NC + lax.axis_index("c")

    pltpu.sync_copy(tab_hbm, tab_v)
    pltpu.sync_copy(tab_hbm.at[pl.ds(0, _K)], pp_v)
    pltpu.sync_copy(tab_hbm.at[pl.ds(_K, _K)], qq_v)
    pltpu.sync_copy(tab_hbm.at[pl.ds(2 * _K, _K)], rr_v)
    pltpu.sync_copy(tab_hbm.at[pl.ds(3 * _K, _K)], ss_v)

    scal = tab_v[pl.ds(4 * _K, _L)]
    c0 = scal[0]
    inv_d = scal[1]
    cmax = scal[2]
    gs = jnp.full((_L,), scal[5], jnp.float32)
    gis = jnp.full((_L,), scal[6], jnp.float32)
    g16s = jnp.full((_L,), scal[7], jnp.float32)
    gi16s = jnp.full((_L,), scal[8], jnp.float32)

    out_bufs = (out0_v, out1_v)
    sems = (sem0, sem1)
    xg_bufs = (xg0_v, xg1_v)
    xsems = (semx0, semx1)
    qg_bufs = (qg0_v, qg1_v)
    qsems = (semq0, semq1)

    base_grp = wid * grp_per_w
    pltpu.async_copy(x_hbm.at[pl.ds(base_grp * _GRP_W, _GRP_W)], xg0_v, semx0)

    def gpair_body(gp, carry):
        for gb in range(2):
            g = gp * 2 + gb
            grp = base_grp + g
            xg_v = xg_bufs[gb]
            qg_v = qg_bufs[gb]

            pltpu.make_async_copy(
                x_hbm.at[pl.ds(0, _GRP_W)], xg_v, xsems[gb]).wait()

            @pl.when(g + 1 < grp_per_w)
            def _():
                pltpu.async_copy(
                    x_hbm.at[pl.ds((grp + 1) * _GRP_W, _GRP_W)],
                    xg_bufs[1 - gb], xsems[1 - gb])

            @pl.when(g >= 2)
            def _():
                pltpu.make_async_copy(
                    qst_hbm.at[pl.ds(0, _GRP_W)], qg_v, qsems[gb]).wait()

            def pair_body(pair, c2, xg_v=xg_v, qg_v=qg_v, g=g, grp=grp):
                for b2 in range(2):
                    wl = pair * 2 + b2
                    out_v = out_bufs[b2]
                    sem = sems[b2]

                    @pl.when(jnp.logical_or(g > 0, pair > 0))
                    def _():
                        pltpu.make_async_copy(
                            assign_hbm.at[pl.ds(0, _PIX_W)], out_v,
                            sem).wait()

                    for chi in range(_CHI):
                        @plsc.parallel_loop(0, 128 // _L, 1, unroll=4)
                        def cv_body(j, chi=chi, wl=wl, out_v=out_v,
                                    xg_v=xg_v, qg_v=qg_v):
                            xoff = chi * 1024 + wl * 128 + j * _L
                            xv = xg_v[pl.ds(xoff, _L)]
                            xc = jnp.minimum(jnp.maximum(xv, c0), cmax)
                            t0 = xc - c0
                            ii = (t0 * inv_d).astype(jnp.int32)
                            p2 = t0 + t0
                            up = jnp.exp(-p2)
                            vp = jnp.exp(p2)
                            pg = plsc.load_gather(pp_v, [ii])
                            qg = plsc.load_gather(qq_v, [ii])
                            rg = plsc.load_gather(rr_v, [ii])
                            sg = plsc.load_gather(ss_v, [ii])
                            inv = 1.0 / (up * pg + vp * qg)
                            qg_v[pl.ds(xoff, _L)] = (up * rg + vp * sg) * inv
                            uas = [up * inv]
                            vbs = [vp * inv]
                            for m in range(3):
                                uas.append(uas[m] * g16s)
                                vbs.append(vbs[m] * gi16s)
                            obase = chi * 1024 + j * _L
                            for t in range(16):
                                for m in range(4):
                                    k = m * 16 + t
                                    e = jnp.minimum(uas[m], vbs[m])
                                    out_v[pl.ds(obase + (k // 8) * 3072
                                                + (k % 8) * 128, _L)] = e
                                    uas[m] = uas[m] * gs
                                    vbs[m] = vbs[m] * gis

                    pltpu.async_copy(
                        out_v,
                        assign_hbm.at[pl.ds((grp * 8 + wl) * _PIX_W, _PIX_W)],
                        sem)
                return c2

            lax.fori_loop(0, 4, pair_body, 0)
            pltpu.async_copy(
                qg_v, qst_hbm.at[pl.ds(grp * _GRP_W, _GRP_W)], qsems[gb])
        return carry

    lax.fori_loop(0, grp_per_w // 2, gpair_body, 0)

    for ov, sm in zip(out_bufs, sems):
        pltpu.make_async_copy(assign_hbm.at[pl.ds(0, _PIX_W)], ov, sm).wait()
    for qv, sm in zip(qg_bufs, qsems):
        pltpu.make_async_copy(qst_hbm.at[pl.ds(0, _GRP_W)], qv, sm).wait()


def kernel(x, centers):
    b, c, h, w = x.shape
    assert c == _C and (b * h * w) % (8 * _NW) == 0, x.shape
    n = x.size

    cf = centers.astype(jnp.float32)
    a_t = jnp.exp(_INV_T * cf)
    b_t = jnp.exp(-_INV_T * cf)
    m4 = jnp.stack([a_t, a_t * cf, b_t, b_t * cf])
    kk = jnp.arange(_K)
    lower = (kk[:, None] <= kk[None, :]).astype(jnp.float32)
    upper = (kk[:, None] > kk[None, :]).astype(jnp.float32)
    w2 = jnp.concatenate([lower, upper], axis=1)
    pr_qs = m4 @ w2
    a0 = a_t[0]
    p_t, r_t = pr_qs[0, :_K] / a0, pr_qs[1, :_K] / a0
    q_t, s_t = pr_qs[2, _K:] * a0, pr_qs[3, _K:] * a0
    inv_d = (_K - 1) / (cf[-1] - cf[0])
    step = (cf[-1] - cf[0]) / (_K - 1)
    g1 = jnp.exp(_INV_T * step)
    scal = jnp.stack([cf[0], inv_d, cf[-1], a_t[0], b_t[0],
                      g1, 1.0 / g1, g1 ** 16, 1.0 / g1 ** 16])
    tab = jnp.concatenate([p_t, q_t, r_t, s_t, scal,
                           jnp.zeros((_K - 9,), jnp.float32)])

    x1d = (x.transpose(0, 2, 3, 1)
            .reshape(b, h, w // 8, 8, _CHI, 128)
            .transpose(0, 1, 2, 4, 3, 5)
            .reshape(-1))

    run = pl.kernel(
        _sq_body,
        out_type=[
            jax.ShapeDtypeStruct((n * _K,), jnp.float32),
            jax.ShapeDtypeStruct((n,), jnp.float32),
        ],
        mesh=plsc.VectorSubcoreMesh(core_axis_name="c", subcore_axis_name="s"),
        compiler_params=pltpu.CompilerParams(needs_layout_passes=False),
        scratch_types=[
            pltpu.VMEM((5 * _K,), jnp.float32),
            pltpu.VMEM((_K,), jnp.float32),
            pltpu.VMEM((_K,), jnp.float32),
            pltpu.VMEM((_K,), jnp.float32),
            pltpu.VMEM((_K,), jnp.float32),
            pltpu.VMEM((_GRP_W,), jnp.float32),
            pltpu.VMEM((_GRP_W,), jnp.float32),
            pltpu.VMEM((_GRP_W,), jnp.float32),
            pltpu.VMEM((_GRP_W,), jnp.float32),
            pltpu.VMEM((_PIX_W,), jnp.float32),
            pltpu.VMEM((_PIX_W,), jnp.float32),
            pltpu.SemaphoreType.DMA,
            pltpu.SemaphoreType.DMA,
            pltpu.SemaphoreType.DMA,
            pltpu.SemaphoreType.DMA,
            pltpu.SemaphoreType.DMA,
            pltpu.SemaphoreType.DMA,
        ],
    )
    assign1d, q1d = run(x1d, tab)

    assign = (assign1d.reshape(b, h, w, 8, _CHI, 8, 128)
              .transpose(0, 4, 6, 1, 2, 3, 5)
              .reshape(b, c, h, w, _K))
    qst = (q1d.reshape(b, h, w // 8, _CHI, 8, 128)
           .transpose(0, 3, 5, 1, 2, 4)
           .reshape(b, c, h, w))
    return qst, assign


# speedup vs baseline: 1.3924x; 1.3924x over previous
import jax
import jax.numpy as jnp
from jax import lax
from jax.experimental import pallas as pl
from jax.experimental.pallas import tpu as pltpu
from jax.experimental.pallas import tpu_sc as plsc

_INV_T = 2.0
_K = 64
_NC, _NS, _L = 2, 16, 16
_NW = _NC * _NS
_C = 384
_CHI = _C // 128
_PIX_W = _K * _CHI * 128
_GRP_W = 8 * _CHI * 128


def _sq_body(x_hbm, tab_hbm, assign_hbm, qst_hbm,
             tab_v, pp_v, qq_v, rr_v, ss_v,
             xg0_v, xg1_v, qg0_v, qg1_v, out0_v, out1_v,
             sem0, sem1, semx0, semx1, semq0, semq1):
    npix = qst_hbm.shape[0] // _C
    grp_per_w = npix // (8 * _NW)
    wid = lax.axis_index("s") * _NC + lax.axis_index("c")

    pltpu.sync_copy(tab_hbm, tab_v)
    pltpu.sync_copy(tab_hbm.at[pl.ds(0, _K)], pp_v)
    pltpu.sync_copy(tab_hbm.at[pl.ds(_K, _K)], qq_v)
    pltpu.sync_copy(tab_hbm.at[pl.ds(2 * _K, _K)], rr_v)
    pltpu.sync_copy(tab_hbm.at[pl.ds(3 * _K, _K)], ss_v)

    scal = tab_v[pl.ds(4 * _K, _L)]
    c0 = scal[0]
    inv_d = scal[1]
    cmax = scal[2]
    gs = jnp.full((_L,), scal[5], jnp.float32)
    gis = jnp.full((_L,), scal[6], jnp.float32)
    g16s = jnp.full((_L,), scal[7], jnp.float32)
    gi16s = jnp.full((_L,), scal[8], jnp.float32)

    out_bufs = (out0_v, out1_v)
    sems = (sem0, sem1)
    xg_bufs = (xg0_v, xg1_v)
    xsems = (semx0, semx1)
    qg_bufs = (qg0_v, qg1_v)
    qsems = (semq0, semq1)

    base_grp = wid * grp_per_w
    pltpu.async_copy(x_hbm.at[pl.ds(base_grp * _GRP_W, _GRP_W)], xg0_v, semx0)

    def gpair_body(gp, carry):
        for gb in range(2):
            g = gp * 2 + gb
            grp = base_grp + g
            xg_v = xg_bufs[gb]
            qg_v = qg_bufs[gb]

            pltpu.make_async_copy(
                x_hbm.at[pl.ds(0, _GRP_W)], xg_v, xsems[gb]).wait()

            @pl.when(g + 1 < grp_per_w)
            def _():
                pltpu.async_copy(
                    x_hbm.at[pl.ds((grp + 1) * _GRP_W, _GRP_W)],
                    xg_bufs[1 - gb], xsems[1 - gb])

            @pl.when(g >= 2)
            def _():
                pltpu.make_async_copy(
                    qst_hbm.at[pl.ds(0, _GRP_W)], qg_v, qsems[gb]).wait()

            def pair_body(pair, c2, xg_v=xg_v, qg_v=qg_v, g=g, grp=grp):
                for b2 in range(2):
                    wl = pair * 2 + b2
                    out_v = out_bufs[b2]
                    sem = sems[b2]

                    @pl.when(jnp.logical_or(g > 0, pair > 0))
                    def _():
                        pltpu.make_async_copy(
                            assign_hbm.at[pl.ds(0, _PIX_W)], out_v,
                            sem).wait()

                    for chi in range(_CHI):
                        @plsc.parallel_loop(0, 128 // _L, 1, unroll=2)
                        def cv_body(j, chi=chi, wl=wl, out_v=out_v,
                                    xg_v=xg_v, qg_v=qg_v):
                            xoff = chi * 1024 + wl * 128 + j * _L
                            xv = xg_v[pl.ds(xoff, _L)]
                            xc = jnp.minimum(jnp.maximum(xv, c0), cmax)
                            t0 = xc - c0
                            ii = (t0 * inv_d).astype(jnp.int32)
                            p2 = t0 + t0
                            up = jnp.exp(-p2)
                            vp = jnp.exp(p2)
                            pg = plsc.load_gather(pp_v, [ii])
                            qg = plsc.load_gather(qq_v, [ii])
                            rg = plsc.load_gather(rr_v, [ii])
                            sg = plsc.load_gather(ss_v, [ii])
                            inv = 1.0 / (up * pg + vp * qg)
                            qg_v[pl.ds(xoff, _L)] = (up * rg + vp * sg) * inv
                            uas = [up * inv]
                            vbs = [vp * inv]
                            for m in range(3):
                                uas.append(uas[m] * g16s)
                                vbs.append(vbs[m] * gi16s)
                            obase = chi * 1024 + j * _L
                            for t in range(16):
                                for m in range(4):
                                    k = m * 16 + t
                                    e = jnp.minimum(uas[m], vbs[m])
                                    out_v[pl.ds(obase + (k // 8) * 3072
                                                + (k % 8) * 128, _L)] = e
                                    uas[m] = uas[m] * gs
                                    vbs[m] = vbs[m] * gis

                    pltpu.async_copy(
                        out_v,
                        assign_hbm.at[pl.ds((grp * 8 + wl) * _PIX_W, _PIX_W)],
                        sem)
                return c2

            lax.fori_loop(0, 4, pair_body, 0)
            pltpu.async_copy(
                qg_v, qst_hbm.at[pl.ds(grp * _GRP_W, _GRP_W)], qsems[gb])
        return carry

    lax.fori_loop(0, grp_per_w // 2, gpair_body, 0)

    for ov, sm in zip(out_bufs, sems):
        pltpu.make_async_copy(assign_hbm.at[pl.ds(0, _PIX_W)], ov, sm).wait()
    for qv, sm in zip(qg_bufs, qsems):
        pltpu.make_async_copy(qst_hbm.at[pl.ds(0, _GRP_W)], qv, sm).wait()


def kernel(x, centers):
    b, c, h, w = x.shape
    assert c == _C and (b * h * w) % (8 * _NW) == 0, x.shape
    n = x.size

    cf = centers.astype(jnp.float32)
    a_t = jnp.exp(_INV_T * cf)
    b_t = jnp.exp(-_INV_T * cf)
    m4 = jnp.stack([a_t, a_t * cf, b_t, b_t * cf])
    kk = jnp.arange(_K)
    lower = (kk[:, None] <= kk[None, :]).astype(jnp.float32)
    upper = (kk[:, None] > kk[None, :]).astype(jnp.float32)
    w2 = jnp.concatenate([lower, upper], axis=1)
    pr_qs = m4 @ w2
    a0 = a_t[0]
    p_t, r_t = pr_qs[0, :_K] / a0, pr_qs[1, :_K] / a0
    q_t, s_t = pr_qs[2, _K:] * a0, pr_qs[3, _K:] * a0
    inv_d = (_K - 1) / (cf[-1] - cf[0])
    step = (cf[-1] - cf[0]) / (_K - 1)
    g1 = jnp.exp(_INV_T * step)
    scal = jnp.stack([cf[0], inv_d, cf[-1], a_t[0], b_t[0],
                      g1, 1.0 / g1, g1 ** 16, 1.0 / g1 ** 16])
    tab = jnp.concatenate([p_t, q_t, r_t, s_t, scal,
                           jnp.zeros((_K - 9,), jnp.float32)])

    x1d = (x.transpose(0, 2, 3, 1)
            .reshape(b, h, w // 8, 8, _CHI, 128)
            .transpose(0, 1, 2, 4, 3, 5)
            .reshape(-1))

    run = pl.kernel(
        _sq_body,
        out_type=[
            jax.ShapeDtypeStruct((n * _K,), jnp.float32),
            jax.ShapeDtypeStruct((n,), jnp.float32),
        ],
        mesh=plsc.VectorSubcoreMesh(core_axis_name="c", subcore_axis_name="s"),
        compiler_params=pltpu.CompilerParams(needs_layout_passes=False),
        scratch_types=[
            pltpu.VMEM((5 * _K,), jnp.float32),
            pltpu.VMEM((_K,), jnp.float32),
            pltpu.VMEM((_K,), jnp.float32),
            pltpu.VMEM((_K,), jnp.float32),
            pltpu.VMEM((_K,), jnp.float32),
            pltpu.VMEM((_GRP_W,), jnp.float32),
            pltpu.VMEM((_GRP_W,), jnp.float32),
            pltpu.VMEM((_GRP_W,), jnp.float32),
            pltpu.VMEM((_GRP_W,), jnp.float32),
            pltpu.VMEM((_PIX_W,), jnp.float32),
            pltpu.VMEM((_PIX_W,), jnp.float32),
            pltpu.SemaphoreType.DMA,
            pltpu.SemaphoreType.DMA,
            pltpu.SemaphoreType.DMA,
            pltpu.SemaphoreType.DMA,
            pltpu.SemaphoreType.DMA,
            pltpu.SemaphoreType.DMA,
        ],
    )
    assign1d, q1d = run(x1d, tab)

    assign = (assign1d.reshape(b, h, w, 8, _CHI, 8, 128)
              .transpose(0, 4, 6, 1, 2, 3, 5)
              .reshape(b, c, h, w, _K))
    qst = (q1d.reshape(b, h, w // 8, _CHI, 8, 128)
           .transpose(0, 3, 5, 1, 2, 4)
           .reshape(b, c, h, w))
    return qst, assign


# speedup vs baseline: 1.4340x; 1.0299x over previous
import jax
import jax.numpy as jnp
from jax import lax
from jax.experimental import pallas as pl
from jax.experimental.pallas import tpu as pltpu
from jax.experimental.pallas import tpu_sc as plsc

_INV_T = 2.0
_K = 64
_NC, _NS, _L = 2, 16, 16
_NW = _NC * _NS
_C = 384
_CHI = _C // 128
_PIX_W = _K * _CHI * 128
_GRP_W = 8 * _CHI * 128


def _sq_body(x_hbm, cen_hbm, assign_hbm, qst_hbm,
             cen_v, pp_v, qq_v, rr_v, ss_v,
             xg0_v, xg1_v, qg0_v, qg1_v, out0_v, out1_v,
             sem0, sem1, semx0, semx1, semq0, semq1):
    npix = qst_hbm.shape[0] // _C
    grp_per_w = npix // (8 * _NW)
    wid = lax.axis_index("s") * _NC + lax.axis_index("c")

    pltpu.sync_copy(cen_hbm, cen_v)

    cb = [cen_v[pl.ds(t * _L, _L)] for t in range(_K // _L)]
    c0 = cb[0][0]
    c0s = jnp.full((_L,), c0, jnp.float32)
    cmaxs = jnp.full((_L,), cb[3][15], jnp.float32)
    step2 = (cb[0][1] - c0) * _INV_T
    gs = jnp.exp(jnp.full((_L,), step2, jnp.float32))
    gis = jnp.exp(jnp.full((_L,), -step2, jnp.float32))
    g16s = jnp.exp(jnp.full((_L,), step2 * 16.0, jnp.float32))
    gi16s = jnp.exp(jnp.full((_L,), step2 * -16.0, jnp.float32))
    inv_dv = (_K - 1.0) / (cmaxs - c0s)
    zero = jnp.zeros((_L,), jnp.float32)
    ca = cac = cbs = cbc = zero
    for t in range(_K // _L):
        d2 = (cb[t] - c0s) * _INV_T
        av = jnp.exp(d2)
        bv = jnp.exp(-d2)
        sa = plsc.cumsum(av) + ca
        sac = plsc.cumsum(av * cb[t]) + cac
        sb = plsc.cumsum(bv) + cbs
        sbc = plsc.cumsum(bv * cb[t]) + cbc
        ca = jnp.full((_L,), sa[_L - 1], jnp.float32)
        cac = jnp.full((_L,), sac[_L - 1], jnp.float32)
        cbs = jnp.full((_L,), sb[_L - 1], jnp.float32)
        cbc = jnp.full((_L,), sbc[_L - 1], jnp.float32)
        pp_v[pl.ds(t * _L, _L)] = sa
        rr_v[pl.ds(t * _L, _L)] = sac
        qq_v[pl.ds(t * _L, _L)] = sb
        ss_v[pl.ds(t * _L, _L)] = sbc
    for t in range(_K // _L):
        qq_v[pl.ds(t * _L, _L)] = cbs - qq_v[pl.ds(t * _L, _L)]
        ss_v[pl.ds(t * _L, _L)] = cbc - ss_v[pl.ds(t * _L, _L)]

    out_bufs = (out0_v, out1_v)
    sems = (sem0, sem1)
    xg_bufs = (xg0_v, xg1_v)
    xsems = (semx0, semx1)
    qg_bufs = (qg0_v, qg1_v)
    qsems = (semq0, semq1)

    base_grp = wid * grp_per_w
    pltpu.async_copy(x_hbm.at[pl.ds(base_grp * _GRP_W, _GRP_W)], xg0_v, semx0)

    def gpair_body(gp, carry):
        for gb in range(2):
            g = gp * 2 + gb
            grp = base_grp + g
            xg_v = xg_bufs[gb]
            qg_v = qg_bufs[gb]

            pltpu.make_async_copy(
                x_hbm.at[pl.ds(0, _GRP_W)], xg_v, xsems[gb]).wait()

            @pl.when(g + 1 < grp_per_w)
            def _():
                pltpu.async_copy(
                    x_hbm.at[pl.ds((grp + 1) * _GRP_W, _GRP_W)],
                    xg_bufs[1 - gb], xsems[1 - gb])

            @pl.when(g >= 2)
            def _():
                pltpu.make_async_copy(
                    qst_hbm.at[pl.ds(0, _GRP_W)], qg_v, qsems[gb]).wait()

            def pair_body(pair, c2, xg_v=xg_v, qg_v=qg_v, g=g, grp=grp):
                for b2 in range(2):
                    wl = pair * 2 + b2
                    out_v = out_bufs[b2]
                    sem = sems[b2]

                    @pl.when(jnp.logical_or(g > 0, pair > 0))
                    def _():
                        pltpu.make_async_copy(
                            assign_hbm.at[pl.ds(0, _PIX_W)], out_v,
                            sem).wait()

                    for chi in range(_CHI):
                        @plsc.parallel_loop(0, 128 // _L, 1, unroll=2)
                        def cv_body(j, chi=chi, wl=wl, out_v=out_v,
                                    xg_v=xg_v, qg_v=qg_v):
                            xoff = chi * 1024 + wl * 128 + j * _L
                            xv = xg_v[pl.ds(xoff, _L)]
                            xc = jnp.minimum(jnp.maximum(xv, c0s), cmaxs)
                            t0 = xc - c0s
                            ii = (t0 * inv_dv).astype(jnp.int32)
                            p2 = t0 + t0
                            up = jnp.exp(-p2)
                            vp = jnp.exp(p2)
                            pg = plsc.load_gather(pp_v, [ii])
                            qg = plsc.load_gather(qq_v, [ii])
                            rg = plsc.load_gather(rr_v, [ii])
                            sg = plsc.load_gather(ss_v, [ii])
                            inv = 1.0 / (up * pg + vp * qg)
                            qg_v[pl.ds(xoff, _L)] = (up * rg + vp * sg) * inv
                            uas = [up * inv]
                            vbs = [vp * inv]
                            for m in range(3):
                                uas.append(uas[m] * g16s)
                                vbs.append(vbs[m] * gi16s)
                            obase = chi * 1024 + j * _L
                            for t in range(16):
                                for m in range(4):
                                    k = m * 16 + t
                                    e = jnp.minimum(uas[m], vbs[m])
                                    out_v[pl.ds(obase + (k // 8) * 3072
                                                + (k % 8) * 128, _L)] = e
                                    uas[m] = uas[m] * gs
                                    vbs[m] = vbs[m] * gis

                    pltpu.async_copy(
                        out_v,
                        assign_hbm.at[pl.ds((grp * 8 + wl) * _PIX_W, _PIX_W)],
                        sem)
                return c2

            lax.fori_loop(0, 4, pair_body, 0)
            pltpu.async_copy(
                qg_v, qst_hbm.at[pl.ds(grp * _GRP_W, _GRP_W)], qsems[gb])
        return carry

    lax.fori_loop(0, grp_per_w // 2, gpair_body, 0)

    for ov, sm in zip(out_bufs, sems):
        pltpu.make_async_copy(assign_hbm.at[pl.ds(0, _PIX_W)], ov, sm).wait()
    for qv, sm in zip(qg_bufs, qsems):
        pltpu.make_async_copy(qst_hbm.at[pl.ds(0, _GRP_W)], qv, sm).wait()


def kernel(x, centers):
    b, c, h, w = x.shape
    assert c == _C and (b * h * w) % (8 * _NW) == 0, x.shape
    n = x.size

    x1d = (x.transpose(0, 2, 3, 1)
            .reshape(b, h, w // 8, 8, _CHI, 128)
            .transpose(0, 1, 2, 4, 3, 5)
            .reshape(-1))

    run = pl.kernel(
        _sq_body,
        out_type=[
            jax.ShapeDtypeStruct((n * _K,), jnp.float32),
            jax.ShapeDtypeStruct((n,), jnp.float32),
        ],
        mesh=plsc.VectorSubcoreMesh(core_axis_name="c", subcore_axis_name="s"),
        compiler_params=pltpu.CompilerParams(needs_layout_passes=False),
        scratch_types=[
            pltpu.VMEM((_K,), jnp.float32),
            pltpu.VMEM((_K,), jnp.float32),
            pltpu.VMEM((_K,), jnp.float32),
            pltpu.VMEM((_K,), jnp.float32),
            pltpu.VMEM((_K,), jnp.float32),
            pltpu.VMEM((_GRP_W,), jnp.float32),
            pltpu.VMEM((_GRP_W,), jnp.float32),
            pltpu.VMEM((_GRP_W,), jnp.float32),
            pltpu.VMEM((_GRP_W,), jnp.float32),
            pltpu.VMEM((_PIX_W,), jnp.float32),
            pltpu.VMEM((_PIX_W,), jnp.float32),
            pltpu.SemaphoreType.DMA,
            pltpu.SemaphoreType.DMA,
            pltpu.SemaphoreType.DMA,
            pltpu.SemaphoreType.DMA,
            pltpu.SemaphoreType.DMA,
            pltpu.SemaphoreType.DMA,
        ],
    )
    assign1d, q1d = run(x1d, centers.astype(jnp.float32))

    assign = (assign1d.reshape(b, h, w, 8, _CHI, 8, 128)
              .transpose(0, 4, 6, 1, 2, 3, 5)
              .reshape(b, c, h, w, _K))
    qst = (q1d.reshape(b, h, w // 8, _CHI, 8, 128)
           .transpose(0, 3, 5, 1, 2, 4)
           .reshape(b, c, h, w))
    return qst, assign
